# baseline (speedup 1.0000x reference)
import functools

import jax
import jax.numpy as jnp
from jax import lax
from jax.experimental import pallas as pl
from jax.experimental.pallas import tpu as pltpu
from jax.experimental.pallas import tpu_sc as plsc

F = 128
NC = 2
NS = 16
NW = NC * NS


def _rows(n, pref):
    return pref if n % pref == 0 else n


def _proj_body(env_ref, w_ref, b_ref, s_ref, d_ref):
    p = jnp.dot(env_ref[...], w_ref[...], preferred_element_type=jnp.float32)
    s_ref[...] = p[:, : 2 * F]
    d_ref[...] = p[:, 2 * F :] + b_ref[...]


def _proj(env, wall, brow):
    n = env.shape[0]
    r = _rows(n, 1000)
    return pl.pallas_call(
        _proj_body,
        grid=(n // r,),
        in_specs=[
            pl.BlockSpec((r, F), lambda i: (i, 0)),
            pl.BlockSpec((F, 4 * F), lambda i: (0, 0)),
            pl.BlockSpec((1, 2 * F), lambda i: (0, 0)),
        ],
        out_specs=[
            pl.BlockSpec((r, 2 * F), lambda i: (i, 0)),
            pl.BlockSpec((r, 2 * F), lambda i: (i, 0)),
        ],
        out_shape=[jax.ShapeDtypeStruct((n, 2 * F), jnp.float32)] * 2,
    )(env, wall, brow)


def _ea_body(ea_ref, w_ref, o1, o2, o3):
    p = jnp.dot(ea_ref[...], w_ref[...], preferred_element_type=jnp.float32)
    o1[...] = p[:, : 2 * F]
    o2[...] = p[:, 2 * F : 4 * F]
    o3[...] = p[:, 4 * F :]


def _ea_proj(ea, w3):
    e, k = ea.shape
    r = _rows(e, 2000)
    return pl.pallas_call(
        _ea_body,
        grid=(e // r,),
        in_specs=[
            pl.BlockSpec((r, k), lambda i: (i, 0)),
            pl.BlockSpec((k, 6 * F), lambda i: (0, 0)),
        ],
        out_specs=[pl.BlockSpec((r, 2 * F), lambda i: (i, 0))] * 3,
        out_shape=[jax.ShapeDtypeStruct((e, 2 * F), jnp.float32)] * 3,
    )(ea, w3)


def _self_msg(s, d):
    gf = s[:, :F] + d[:, :F]
    gs = s[:, F:] + d[:, F:]
    sig = 1.0 / (1.0 + jnp.exp(-gf))
    sp = jnp.maximum(gs, 0.0) + jnp.log(1.0 + jnp.exp(-jnp.abs(gs)))
    return sig * sp


def _update_body(e_ref, p_ref, s_ref, d_ref, o_ref):
    o_ref[...] = (
        e_ref[...]
        + p_ref[0]
        + p_ref[1]
        + _self_msg(s_ref[...], d_ref[...])
    )


def _update(env, part, s, d):
    n = env.shape[0]
    r = _rows(n, 1000)
    return pl.pallas_call(
        _update_body,
        grid=(n // r,),
        in_specs=[
            pl.BlockSpec((r, F), lambda i: (i, 0)),
            pl.BlockSpec((NC, r, F), lambda i: (0, i, 0)),
            pl.BlockSpec((r, 2 * F), lambda i: (i, 0)),
            pl.BlockSpec((r, 2 * F), lambda i: (i, 0)),
        ],
        out_specs=pl.BlockSpec((r, F), lambda i: (i, 0)),
        out_shape=jax.ShapeDtypeStruct((n, F), jnp.float32),
    )(env, part, s, d)


def _final_body(e_ref, p_ref, s_ref, d_ref, w21_ref, b21_ref, w22_ref,
                b22_ref, w23_ref, b23_ref, o_ref, acc_ref, *, n):
    i = pl.program_id(0)

    @pl.when(i == 0)
    def _():
        acc_ref[...] = jnp.zeros_like(acc_ref)

    env4 = (
        e_ref[...]
        + p_ref[0]
        + p_ref[1]
        + _self_msg(s_ref[...], d_ref[...])
    )
    acc_ref[...] += jnp.sum(env4, axis=0, keepdims=True)

    @pl.when(i == pl.num_programs(0) - 1)
    def _():
        pooled = acc_ref[...] * (1.0 / n)
        h1 = jnp.maximum(
            jnp.dot(pooled, w21_ref[...], preferred_element_type=jnp.float32)
            + b21_ref[...], 0.0)
        h2 = jnp.maximum(
            jnp.dot(h1, w22_ref[...], preferred_element_type=jnp.float32)
            + b22_ref[...], 0.0)
        o_ref[...] = (
            jnp.sum(h2 * w23_ref[...], axis=1, keepdims=True) + b23_ref[...]
        )


def _final(env, part, s, d, w21, b21, w22, b22, w23r, b23r):
    n = env.shape[0]
    r = _rows(n, 1000)
    return pl.pallas_call(
        functools.partial(_final_body, n=n),
        grid=(n // r,),
        in_specs=[
            pl.BlockSpec((r, F), lambda i: (i, 0)),
            pl.BlockSpec((NC, r, F), lambda i: (0, i, 0)),
            pl.BlockSpec((r, 2 * F), lambda i: (i, 0)),
            pl.BlockSpec((r, 2 * F), lambda i: (i, 0)),
            pl.BlockSpec((F, 2 * F), lambda i: (0, 0)),
            pl.BlockSpec((1, 2 * F), lambda i: (0, 0)),
            pl.BlockSpec((2 * F, F), lambda i: (0, 0)),
            pl.BlockSpec((1, F), lambda i: (0, 0)),
            pl.BlockSpec((1, F), lambda i: (0, 0)),
            pl.BlockSpec((1, 1), lambda i: (0, 0)),
        ],
        out_specs=pl.BlockSpec((1, 1), lambda i: (0, 0)),
        out_shape=jax.ShapeDtypeStruct((1, 1), jnp.float32),
        scratch_shapes=[pltpu.VMEM((1, F), jnp.float32)],
        compiler_params=pltpu.CompilerParams(
            dimension_semantics=("arbitrary",)),
    )(env, part, s, d, w21, b21, w22, b22, w23r, b23r)


_B = 40


def _sc_body(s_hbm, d_hbm, ea_hbm, src_hbm, dst_hbm, z_hbm, out_hbm,
             idx_s, idx_d, buf_s, buf_d, buf_e, buf_m, acc,
             sg1, sg2, sg3, *, n, e):
    c = lax.axis_index("c")
    s = lax.axis_index("s")
    wid = s * NC + c
    epw = e // NW
    chunks = epw // _B
    rpt = (n // NS) // 8 * 8
    rem = n - NS * rpt

    r0 = pl.multiple_of(s * rpt, 8)
    pltpu.sync_copy(z_hbm.at[pl.ds(r0, rpt)], acc.at[pl.ds(r0, rpt)])
    if rem:
        @pl.when(s == 0)
        def _():
            pltpu.sync_copy(z_hbm.at[pl.ds(NS * rpt, rem)],
                            acc.at[pl.ds(NS * rpt, rem)])
    plsc.subcore_barrier()

    base = wid * epw

    def chunk(i, carry):
        off = pl.multiple_of(base + i * _B, 8)
        pltpu.sync_copy(src_hbm.at[pl.ds(off, _B)], idx_s)
        pltpu.sync_copy(dst_hbm.at[pl.ds(off, _B)], idx_d)
        g1 = pltpu.async_copy(s_hbm.at[idx_s], buf_s, sg1)
        g2 = pltpu.async_copy(d_hbm.at[idx_d], buf_d, sg2)
        g3 = pltpu.async_copy(ea_hbm.at[pl.ds(off, _B)], buf_e, sg3)
        g1.wait()
        g2.wait()
        g3.wait()

        def row(rr, carry2):
            for cc in range(F // 16):
                o = cc * 16
                gf = (buf_s[rr, pl.ds(o, 16)] + buf_d[rr, pl.ds(o, 16)]
                      + buf_e[rr, pl.ds(o, 16)])
                gs = (buf_s[rr, pl.ds(F + o, 16)] + buf_d[rr, pl.ds(F + o, 16)]
                      + buf_e[rr, pl.ds(F + o, 16)])
                sig = 1.0 / (1.0 + jnp.exp(-gf))
                t = jnp.exp(-jnp.abs(gs))
                rr_ = t / (2.0 + t)
                r2 = rr_ * rr_
                l1p = 2.0 * rr_ * (1.0 + r2 * (
                    (1.0 / 3.0) + r2 * ((1.0 / 5.0) + r2 * (
                        (1.0 / 7.0) + r2 * (1.0 / 9.0)))))
                sp = jnp.maximum(gs, 0.0) + l1p
                buf_m[rr, pl.ds(o, 16)] = sig * sp
            return carry2

        lax.fori_loop(0, _B, row, 0)
        pltpu.sync_copy(buf_m, acc.at[idx_d], add=True)
        return carry

    lax.fori_loop(0, chunks, chunk, 0)
    plsc.subcore_barrier()
    pltpu.sync_copy(acc.at[pl.ds(r0, rpt)], out_hbm.at[c, pl.ds(r0, rpt)])
    if rem:
        @pl.when(s == 0)
        def _():
            pltpu.sync_copy(acc.at[pl.ds(NS * rpt, rem)],
                            out_hbm.at[c, pl.ds(NS * rpt, rem)])


def _sc_edge(s, d, ea, src, dst, zeros):
    n = s.shape[0]
    e = src.shape[0]
    mesh = plsc.VectorSubcoreMesh(
        core_axis_name="c", subcore_axis_name="s",
        num_cores=NC, num_subcores=NS)
    kern = pl.kernel(
        functools.partial(_sc_body, n=n, e=e),
        out_type=jax.ShapeDtypeStruct((NC, n, F), jnp.float32),
        mesh=mesh,
        scratch_types=[
            pltpu.VMEM((_B,), jnp.int32),
            pltpu.VMEM((_B,), jnp.int32),
            pltpu.VMEM((_B, 2 * F), jnp.float32),
            pltpu.VMEM((_B, 2 * F), jnp.float32),
            pltpu.VMEM((_B, 2 * F), jnp.float32),
            pltpu.VMEM((_B, F), jnp.float32),
            pltpu.VMEM_SHARED((n, F), jnp.float32),
            pltpu.SemaphoreType.DMA,
            pltpu.SemaphoreType.DMA,
            pltpu.SemaphoreType.DMA,
        ],
    )
    return kern(s, d, ea, src, dst, zeros)


def kernel(x, edge_index, edge_attr, wf1, bf1, ws1, bs1, wf2, bf2, ws2, bs2,
           wf3, bf3, ws3, bs3, w21, b21, w22, b22, w23, b23):
    n = x.shape[0]
    src = edge_index[0]
    dst = edge_index[1]
    zeros = jnp.zeros((n, F), jnp.float32)

    def mk(wf, ws, bf, bs):
        wall = jnp.concatenate(
            [wf[:F], ws[:F], wf[F:2 * F], ws[F:2 * F]], axis=1)
        brow = jnp.concatenate([bf, bs]).reshape(1, 2 * F)
        wea = jnp.concatenate([wf[2 * F:], ws[2 * F:]], axis=1)
        return wall, brow, wea

    wall1, brow1, wea1 = mk(wf1, ws1, bf1, bs1)
    wall2, brow2, wea2 = mk(wf2, ws2, bf2, bs2)
    wall3, brow3, wea3 = mk(wf3, ws3, bf3, bs3)

    ea1, ea2, ea3 = _ea_proj(
        edge_attr, jnp.concatenate([wea1, wea2, wea3], axis=1))

    s1, d1 = _proj(x, wall1, brow1)
    p1 = _sc_edge(s1, d1, ea1, src, dst, zeros)
    env2 = _update(x, p1, s1, d1)

    s2, d2 = _proj(env2, wall2, brow2)
    p2 = _sc_edge(s2, d2, ea2, src, dst, zeros)
    env3 = _update(env2, p2, s2, d2)

    s3, d3 = _proj(env3, wall3, brow3)
    p3 = _sc_edge(s3, d3, ea3, src, dst, zeros)

    return _final(env3, p3, s3, d3,
                  w21, b21.reshape(1, 2 * F),
                  w22, b22.reshape(1, F),
                  w23.reshape(1, F), b23.reshape(1, 1))


# speedup vs baseline: 1.0245x; 1.0245x over previous
import functools

import jax
import jax.numpy as jnp
from jax import lax
from jax.experimental import pallas as pl
from jax.experimental.pallas import tpu as pltpu
from jax.experimental.pallas import tpu_sc as plsc

F = 128
NC = 2
NS = 16
NW = NC * NS


def _rows(n, pref):
    return pref if n % pref == 0 else n


def _proj_body(env_ref, w_ref, b_ref, s_ref, d_ref):
    p = jnp.dot(env_ref[...], w_ref[...], preferred_element_type=jnp.float32)
    s_ref[...] = p[:, : 2 * F]
    d_ref[...] = p[:, 2 * F :] + b_ref[...]


def _proj(env, wall, brow):
    n = env.shape[0]
    r = _rows(n, 1000)
    return pl.pallas_call(
        _proj_body,
        grid=(n // r,),
        in_specs=[
            pl.BlockSpec((r, F), lambda i: (i, 0)),
            pl.BlockSpec((F, 4 * F), lambda i: (0, 0)),
            pl.BlockSpec((1, 2 * F), lambda i: (0, 0)),
        ],
        out_specs=[
            pl.BlockSpec((r, 2 * F), lambda i: (i, 0)),
            pl.BlockSpec((r, 2 * F), lambda i: (i, 0)),
        ],
        out_shape=[jax.ShapeDtypeStruct((n, 2 * F), jnp.float32)] * 2,
    )(env, wall, brow)


def _ea_body(ea_ref, w_ref, o1, o2, o3):
    p = jnp.dot(ea_ref[...], w_ref[...], preferred_element_type=jnp.float32)
    o1[...] = p[:, : 2 * F]
    o2[...] = p[:, 2 * F : 4 * F]
    o3[...] = p[:, 4 * F :]


def _ea_proj(ea, w3):
    e, k = ea.shape
    r = _rows(e, 2000)
    return pl.pallas_call(
        _ea_body,
        grid=(e // r,),
        in_specs=[
            pl.BlockSpec((r, k), lambda i: (i, 0)),
            pl.BlockSpec((k, 6 * F), lambda i: (0, 0)),
        ],
        out_specs=[pl.BlockSpec((r, 2 * F), lambda i: (i, 0))] * 3,
        out_shape=[jax.ShapeDtypeStruct((e, 2 * F), jnp.float32)] * 3,
    )(ea, w3)


def _self_msg(s, d):
    gf = s[:, :F] + d[:, :F]
    gs = s[:, F:] + d[:, F:]
    sig = 1.0 / (1.0 + jnp.exp(-gf))
    sp = jnp.maximum(gs, 0.0) + jnp.log(1.0 + jnp.exp(-jnp.abs(gs)))
    return sig * sp


def _update_body(e_ref, p_ref, s_ref, d_ref, o_ref):
    o_ref[...] = (
        e_ref[...]
        + p_ref[0]
        + p_ref[1]
        + _self_msg(s_ref[...], d_ref[...])
    )


def _update(env, part, s, d):
    n = env.shape[0]
    r = _rows(n, 1000)
    return pl.pallas_call(
        _update_body,
        grid=(n // r,),
        in_specs=[
            pl.BlockSpec((r, F), lambda i: (i, 0)),
            pl.BlockSpec((NC, r, F), lambda i: (0, i, 0)),
            pl.BlockSpec((r, 2 * F), lambda i: (i, 0)),
            pl.BlockSpec((r, 2 * F), lambda i: (i, 0)),
        ],
        out_specs=pl.BlockSpec((r, F), lambda i: (i, 0)),
        out_shape=jax.ShapeDtypeStruct((n, F), jnp.float32),
    )(env, part, s, d)


def _final_body(e_ref, p_ref, s_ref, d_ref, w21_ref, b21_ref, w22_ref,
                b22_ref, w23_ref, b23_ref, o_ref, acc_ref, *, n):
    i = pl.program_id(0)

    @pl.when(i == 0)
    def _():
        acc_ref[...] = jnp.zeros_like(acc_ref)

    env4 = (
        e_ref[...]
        + p_ref[0]
        + p_ref[1]
        + _self_msg(s_ref[...], d_ref[...])
    )
    acc_ref[...] += jnp.sum(env4, axis=0, keepdims=True)

    @pl.when(i == pl.num_programs(0) - 1)
    def _():
        pooled = acc_ref[...] * (1.0 / n)
        h1 = jnp.maximum(
            jnp.dot(pooled, w21_ref[...], preferred_element_type=jnp.float32)
            + b21_ref[...], 0.0)
        h2 = jnp.maximum(
            jnp.dot(h1, w22_ref[...], preferred_element_type=jnp.float32)
            + b22_ref[...], 0.0)
        o_ref[...] = (
            jnp.sum(h2 * w23_ref[...], axis=1, keepdims=True) + b23_ref[...]
        )


def _final(env, part, s, d, w21, b21, w22, b22, w23r, b23r):
    n = env.shape[0]
    r = _rows(n, 1000)
    return pl.pallas_call(
        functools.partial(_final_body, n=n),
        grid=(n // r,),
        in_specs=[
            pl.BlockSpec((r, F), lambda i: (i, 0)),
            pl.BlockSpec((NC, r, F), lambda i: (0, i, 0)),
            pl.BlockSpec((r, 2 * F), lambda i: (i, 0)),
            pl.BlockSpec((r, 2 * F), lambda i: (i, 0)),
            pl.BlockSpec((F, 2 * F), lambda i: (0, 0)),
            pl.BlockSpec((1, 2 * F), lambda i: (0, 0)),
            pl.BlockSpec((2 * F, F), lambda i: (0, 0)),
            pl.BlockSpec((1, F), lambda i: (0, 0)),
            pl.BlockSpec((1, F), lambda i: (0, 0)),
            pl.BlockSpec((1, 1), lambda i: (0, 0)),
        ],
        out_specs=pl.BlockSpec((1, 1), lambda i: (0, 0)),
        out_shape=jax.ShapeDtypeStruct((1, 1), jnp.float32),
        scratch_shapes=[pltpu.VMEM((1, F), jnp.float32)],
        compiler_params=pltpu.CompilerParams(
            dimension_semantics=("arbitrary",)),
    )(env, part, s, d, w21, b21, w22, b22, w23r, b23r)


_B = 16


def _gate16(bs, bd, be, bm, rr):
    for cc in range(F // 16):
        o = cc * 16
        gf = (bs[rr, pl.ds(o, 16)] + bd[rr, pl.ds(o, 16)]
              + be[rr, pl.ds(o, 16)])
        gs = (bs[rr, pl.ds(F + o, 16)] + bd[rr, pl.ds(F + o, 16)]
              + be[rr, pl.ds(F + o, 16)])
        sig = 1.0 / (1.0 + jnp.exp(-gf))
        t = jnp.exp(-jnp.abs(gs))
        r_ = t / (2.0 + t)
        r2 = r_ * r_
        l1p = 2.0 * r_ * (1.0 + r2 * (
            (1.0 / 3.0) + r2 * ((1.0 / 5.0) + r2 * (
                (1.0 / 7.0) + r2 * (1.0 / 9.0)))))
        sp = jnp.maximum(gs, 0.0) + l1p
        bm[rr, pl.ds(o, 16)] = sig * sp


def _sc_body(s_hbm, d_hbm, ea_hbm, src_hbm, dst_hbm, z_hbm, out_hbm,
             idx_s0, idx_d0, idx_s1, idx_d1,
             bs0, bd0, be0, bm0, bs1, bd1, be1, bm1,
             acc, ss0, sd0, se0, ss1, sd1, se1, *, n, e):
    c = lax.axis_index("c")
    s = lax.axis_index("s")
    wid = s * NC + c
    epw = e // NW
    chunks = epw // _B
    rpt = (n // NS) // 8 * 8
    rem = n - NS * rpt
    r0 = pl.multiple_of(s * rpt, 8)
    pltpu.sync_copy(z_hbm.at[pl.ds(r0, rpt)], acc.at[pl.ds(r0, rpt)])
    if rem:
        @pl.when(s == 0)
        def _():
            pltpu.sync_copy(z_hbm.at[pl.ds(NS * rpt, rem)],
                            acc.at[pl.ds(NS * rpt, rem)])
    plsc.subcore_barrier()

    base = pl.multiple_of(wid * epw, 8)

    parity = ((idx_s0, idx_d0, bs0, bd0, be0, bm0, ss0, sd0, se0),
              (idx_s1, idx_d1, bs1, bd1, be1, bm1, ss1, sd1, se1))

    def issue(g, p):
        ixs, ixd, bs, bd, be, _, s_s, s_d, s_e = parity[p]
        off = pl.multiple_of(base + g * _B, 8)
        pltpu.sync_copy(src_hbm.at[pl.ds(off, _B)], ixs)
        pltpu.sync_copy(dst_hbm.at[pl.ds(off, _B)], ixd)
        pltpu.async_copy(s_hbm.at[ixs], bs, s_s)
        pltpu.async_copy(d_hbm.at[ixd], bd, s_d)
        pltpu.async_copy(ea_hbm.at[pl.ds(off, _B)], be, s_e)

    def consume(g, p):
        ixs, ixd, bs, bd, be, bm, s_s, s_d, s_e = parity[p]
        pltpu.make_async_copy(s_hbm.at[ixs], bs, s_s).wait()
        pltpu.make_async_copy(d_hbm.at[ixd], bd, s_d).wait()
        pltpu.make_async_copy(ea_hbm.at[pl.ds(0, _B)], be, s_e).wait()

        def _row(rr, carry2):
            _gate16(bs, bd, be, bm, rr)
            return carry2

        lax.fori_loop(0, _B, _row, 0)

        pltpu.sync_copy(bm, acc.at[ixd], add=True)

    issue(0, 0)

    def pair(k, carry):
        issue(2 * k + 1, 1)
        consume(2 * k, 0)
        issue(2 * k + 2, 0)
        consume(2 * k + 1, 1)
        return carry

    lax.fori_loop(0, (chunks - 1) // 2, pair, 0)
    consume(chunks - 1, 0)

    plsc.subcore_barrier()
    pltpu.sync_copy(acc.at[pl.ds(r0, rpt)], out_hbm.at[c, pl.ds(r0, rpt)])
    if rem:
        @pl.when(s == 0)
        def _():
            pltpu.sync_copy(acc.at[pl.ds(NS * rpt, rem)],
                            out_hbm.at[c, pl.ds(NS * rpt, rem)])


def _sc_edge(s, d, ea, src, dst, zeros):
    n = s.shape[0]
    e = src.shape[0]
    epw = e // NW
    mesh = plsc.VectorSubcoreMesh(
        core_axis_name="c", subcore_axis_name="s",
        num_cores=NC, num_subcores=NS)
    kern = pl.kernel(
        functools.partial(_sc_body, n=n, e=e),
        out_type=jax.ShapeDtypeStruct((NC, n, F), jnp.float32),
        mesh=mesh,
        scratch_types=[
            pltpu.VMEM((_B,), jnp.int32),
            pltpu.VMEM((_B,), jnp.int32),
            pltpu.VMEM((_B,), jnp.int32),
            pltpu.VMEM((_B,), jnp.int32),
            pltpu.VMEM((_B, 2 * F), jnp.float32),
            pltpu.VMEM((_B, 2 * F), jnp.float32),
            pltpu.VMEM((_B, 2 * F), jnp.float32),
            pltpu.VMEM((_B, F), jnp.float32),
            pltpu.VMEM((_B, 2 * F), jnp.float32),
            pltpu.VMEM((_B, 2 * F), jnp.float32),
            pltpu.VMEM((_B, 2 * F), jnp.float32),
            pltpu.VMEM((_B, F), jnp.float32),
            pltpu.VMEM_SHARED((n, F), jnp.float32),
            pltpu.SemaphoreType.DMA,
            pltpu.SemaphoreType.DMA,
            pltpu.SemaphoreType.DMA,
            pltpu.SemaphoreType.DMA,
            pltpu.SemaphoreType.DMA,
            pltpu.SemaphoreType.DMA,
        ],
    )
    return kern(s, d, ea, src, dst, zeros)


def kernel(x, edge_index, edge_attr, wf1, bf1, ws1, bs1, wf2, bf2, ws2, bs2,
           wf3, bf3, ws3, bs3, w21, b21, w22, b22, w23, b23):
    n = x.shape[0]
    src = edge_index[0]
    dst = edge_index[1]
    zeros = jnp.zeros((n, F), jnp.float32)

    def mk(wf, ws, bf, bs):
        wall = jnp.concatenate(
            [wf[:F], ws[:F], wf[F:2 * F], ws[F:2 * F]], axis=1)
        brow = jnp.concatenate([bf, bs]).reshape(1, 2 * F)
        wea = jnp.concatenate([wf[2 * F:], ws[2 * F:]], axis=1)
        return wall, brow, wea

    wall1, brow1, wea1 = mk(wf1, ws1, bf1, bs1)
    wall2, brow2, wea2 = mk(wf2, ws2, bf2, bs2)
    wall3, brow3, wea3 = mk(wf3, ws3, bf3, bs3)

    ea1, ea2, ea3 = _ea_proj(
        edge_attr, jnp.concatenate([wea1, wea2, wea3], axis=1))

    s1, d1 = _proj(x, wall1, brow1)
    p1 = _sc_edge(s1, d1, ea1, src, dst, zeros)
    env2 = _update(x, p1, s1, d1)

    s2, d2 = _proj(env2, wall2, brow2)
    p2 = _sc_edge(s2, d2, ea2, src, dst, zeros)
    env3 = _update(env2, p2, s2, d2)

    s3, d3 = _proj(env3, wall3, brow3)
    p3 = _sc_edge(s3, d3, ea3, src, dst, zeros)

    return _final(env3, p3, s3, d3,
                  w21, b21.reshape(1, 2 * F),
                  w22, b22.reshape(1, F),
                  w23.reshape(1, F), b23.reshape(1, 1))


# speedup vs baseline: 1.1614x; 1.1336x over previous
import functools

import jax
import jax.numpy as jnp
from jax import lax
from jax.experimental import pallas as pl
from jax.experimental.pallas import tpu as pltpu
from jax.experimental.pallas import tpu_sc as plsc

F = 128
NC = 2
NS = 16
NW = NC * NS


def _rows(n, pref):
    return pref if n % pref == 0 else n


def _proj_body(env_ref, w_ref, b_ref, s_ref, d_ref):
    p = jnp.dot(env_ref[...], w_ref[...], preferred_element_type=jnp.float32)
    s_ref[...] = p[:, : 2 * F]
    d_ref[...] = p[:, 2 * F :] + b_ref[...]


def _proj(env, wall, brow):
    n = env.shape[0]
    r = _rows(n, 1000)
    return pl.pallas_call(
        _proj_body,
        grid=(n // r,),
        in_specs=[
            pl.BlockSpec((r, F), lambda i: (i, 0)),
            pl.BlockSpec((F, 4 * F), lambda i: (0, 0)),
            pl.BlockSpec((1, 2 * F), lambda i: (0, 0)),
        ],
        out_specs=[
            pl.BlockSpec((r, 2 * F), lambda i: (i, 0)),
            pl.BlockSpec((r, 2 * F), lambda i: (i, 0)),
        ],
        out_shape=[jax.ShapeDtypeStruct((n, 2 * F), jnp.float32)] * 2,
    )(env, wall, brow)


def _ea_body(ea_ref, w_ref, o1, o2, o3):
    p = jnp.dot(ea_ref[...], w_ref[...], preferred_element_type=jnp.float32)
    o1[...] = p[:, : 2 * F]
    o2[...] = p[:, 2 * F : 4 * F]
    o3[...] = p[:, 4 * F :]


def _ea_proj(ea, w3):
    e, k = ea.shape
    r = _rows(e, 2000)
    return pl.pallas_call(
        _ea_body,
        grid=(e // r,),
        in_specs=[
            pl.BlockSpec((r, k), lambda i: (i, 0)),
            pl.BlockSpec((k, 6 * F), lambda i: (0, 0)),
        ],
        out_specs=[pl.BlockSpec((r, 2 * F), lambda i: (i, 0))] * 3,
        out_shape=[jax.ShapeDtypeStruct((e, 2 * F), jnp.float32)] * 3,
    )(ea, w3)


def _self_msg(s, d):
    gf = s[:, :F] + d[:, :F]
    gs = s[:, F:] + d[:, F:]
    sig = 1.0 / (1.0 + jnp.exp(-gf))
    sp = jnp.maximum(gs, 0.0) + jnp.log(1.0 + jnp.exp(-jnp.abs(gs)))
    return sig * sp


def _update_body(e_ref, p_ref, s_ref, d_ref, o_ref):
    o_ref[...] = (
        e_ref[...]
        + p_ref[0]
        + p_ref[1]
        + _self_msg(s_ref[...], d_ref[...])
    )


def _update(env, part, s, d):
    n = env.shape[0]
    r = _rows(n, 1000)
    return pl.pallas_call(
        _update_body,
        grid=(n // r,),
        in_specs=[
            pl.BlockSpec((r, F), lambda i: (i, 0)),
            pl.BlockSpec((NC, r, F), lambda i: (0, i, 0)),
            pl.BlockSpec((r, 2 * F), lambda i: (i, 0)),
            pl.BlockSpec((r, 2 * F), lambda i: (i, 0)),
        ],
        out_specs=pl.BlockSpec((r, F), lambda i: (i, 0)),
        out_shape=jax.ShapeDtypeStruct((n, F), jnp.float32),
    )(env, part, s, d)


def _final_body(e_ref, p_ref, s_ref, d_ref, w21_ref, b21_ref, w22_ref,
                b22_ref, w23_ref, b23_ref, o_ref, acc_ref, *, n):
    i = pl.program_id(0)

    @pl.when(i == 0)
    def _():
        acc_ref[...] = jnp.zeros_like(acc_ref)

    env4 = (
        e_ref[...]
        + p_ref[0]
        + p_ref[1]
        + _self_msg(s_ref[...], d_ref[...])
    )
    acc_ref[...] += jnp.sum(env4, axis=0, keepdims=True)

    @pl.when(i == pl.num_programs(0) - 1)
    def _():
        pooled = acc_ref[...] * (1.0 / n)
        h1 = jnp.maximum(
            jnp.dot(pooled, w21_ref[...], preferred_element_type=jnp.float32)
            + b21_ref[...], 0.0)
        h2 = jnp.maximum(
            jnp.dot(h1, w22_ref[...], preferred_element_type=jnp.float32)
            + b22_ref[...], 0.0)
        o_ref[...] = (
            jnp.sum(h2 * w23_ref[...], axis=1, keepdims=True) + b23_ref[...]
        )


def _final(env, part, s, d, w21, b21, w22, b22, w23r, b23r):
    n = env.shape[0]
    r = _rows(n, 1000)
    return pl.pallas_call(
        functools.partial(_final_body, n=n),
        grid=(n // r,),
        in_specs=[
            pl.BlockSpec((r, F), lambda i: (i, 0)),
            pl.BlockSpec((NC, r, F), lambda i: (0, i, 0)),
            pl.BlockSpec((r, 2 * F), lambda i: (i, 0)),
            pl.BlockSpec((r, 2 * F), lambda i: (i, 0)),
            pl.BlockSpec((F, 2 * F), lambda i: (0, 0)),
            pl.BlockSpec((1, 2 * F), lambda i: (0, 0)),
            pl.BlockSpec((2 * F, F), lambda i: (0, 0)),
            pl.BlockSpec((1, F), lambda i: (0, 0)),
            pl.BlockSpec((1, F), lambda i: (0, 0)),
            pl.BlockSpec((1, 1), lambda i: (0, 0)),
        ],
        out_specs=pl.BlockSpec((1, 1), lambda i: (0, 0)),
        out_shape=jax.ShapeDtypeStruct((1, 1), jnp.float32),
        scratch_shapes=[pltpu.VMEM((1, F), jnp.float32)],
        compiler_params=pltpu.CompilerParams(
            dimension_semantics=("arbitrary",)),
    )(env, part, s, d, w21, b21, w22, b22, w23r, b23r)


_B = 16


def _gate16(bs, bd, be, bm, rr):
    for cc in range(F // 16):
        o = cc * 16
        gf = (bs[rr, pl.ds(o, 16)] + bd[rr, pl.ds(o, 16)]
              + be[rr, pl.ds(o, 16)])
        gs = (bs[rr, pl.ds(F + o, 16)] + bd[rr, pl.ds(F + o, 16)]
              + be[rr, pl.ds(F + o, 16)])
        sig = 1.0 / (1.0 + jnp.exp(-gf))
        t = jnp.exp(-jnp.abs(gs))
        r_ = t / (2.0 + t)
        r2 = r_ * r_
        l1p = 2.0 * r_ * (1.0 + r2 * (
            (1.0 / 3.0) + r2 * ((1.0 / 5.0) + r2 * (
                (1.0 / 7.0) + r2 * (1.0 / 9.0)))))
        sp = jnp.maximum(gs, 0.0) + l1p
        bm[rr, pl.ds(o, 16)] = sig * sp


def _sc_body(s_hbm, d_hbm, ea_hbm, src_hbm, dst_hbm, z_hbm, out_hbm,
             src_all, dst_all, idx_s0, idx_d0, idx_s1, idx_d1,
             bs0, bd0, be0, bm0, bs1, bd1, be1, bm1,
             acc, ss0, sd0, se0, ss1, sd1, se1, *, n, e):
    c = lax.axis_index("c")
    s = lax.axis_index("s")
    wid = s * NC + c
    epw = e // NW
    chunks = epw // _B
    rpt = (n // NS) // 8 * 8
    rem = n - NS * rpt
    r0 = pl.multiple_of(s * rpt, 8)
    pltpu.sync_copy(z_hbm.at[pl.ds(r0, rpt)], acc.at[pl.ds(r0, rpt)])
    if rem:
        @pl.when(s == 0)
        def _():
            pltpu.sync_copy(z_hbm.at[pl.ds(NS * rpt, rem)],
                            acc.at[pl.ds(NS * rpt, rem)])
    plsc.subcore_barrier()

    base = pl.multiple_of(wid * epw, 8)
    pltpu.sync_copy(src_hbm.at[pl.ds(base, epw)], src_all)
    pltpu.sync_copy(dst_hbm.at[pl.ds(base, epw)], dst_all)

    parity = ((idx_s0, idx_d0, bs0, bd0, be0, bm0, ss0, sd0, se0),
              (idx_s1, idx_d1, bs1, bd1, be1, bm1, ss1, sd1, se1))

    def issue(g, p):
        _, _, bs, bd, be, _, s_s, s_d, s_e = parity[p]
        io = pl.multiple_of(g * _B, 8)
        pltpu.async_copy(s_hbm.at[src_all[pl.ds(io, _B)]], bs, s_s)
        pltpu.async_copy(d_hbm.at[dst_all[pl.ds(io, _B)]], bd, s_d)
        off = pl.multiple_of(base + g * _B, 8)
        pltpu.async_copy(ea_hbm.at[pl.ds(off, _B)], be, s_e)

    def consume(g, p):
        ixs, ixd, bs, bd, be, bm, s_s, s_d, s_e = parity[p]
        io = pl.multiple_of(g * _B, 8)
        pltpu.make_async_copy(s_hbm.at[src_all[pl.ds(io, _B)]], bs, s_s).wait()
        pltpu.make_async_copy(d_hbm.at[dst_all[pl.ds(io, _B)]], bd, s_d).wait()
        pltpu.make_async_copy(ea_hbm.at[pl.ds(0, _B)], be, s_e).wait()

        def _row(rr, carry2):
            _gate16(bs, bd, be, bm, rr)
            return carry2

        lax.fori_loop(0, _B, _row, 0)

        pltpu.sync_copy(bm, acc.at[dst_all[pl.ds(io, _B)]], add=True)

    issue(0, 0)

    def pair(k, carry):
        issue(2 * k + 1, 1)
        consume(2 * k, 0)
        issue(2 * k + 2, 0)
        consume(2 * k + 1, 1)
        return carry

    lax.fori_loop(0, (chunks - 1) // 2, pair, 0)
    consume(chunks - 1, 0)

    plsc.subcore_barrier()
    pltpu.sync_copy(acc.at[pl.ds(r0, rpt)], out_hbm.at[c, pl.ds(r0, rpt)])
    if rem:
        @pl.when(s == 0)
        def _():
            pltpu.sync_copy(acc.at[pl.ds(NS * rpt, rem)],
                            out_hbm.at[c, pl.ds(NS * rpt, rem)])


def _sc_edge(s, d, ea, src, dst, zeros):
    n = s.shape[0]
    e = src.shape[0]
    epw = e // NW
    mesh = plsc.VectorSubcoreMesh(
        core_axis_name="c", subcore_axis_name="s",
        num_cores=NC, num_subcores=NS)
    kern = pl.kernel(
        functools.partial(_sc_body, n=n, e=e),
        out_type=jax.ShapeDtypeStruct((NC, n, F), jnp.float32),
        mesh=mesh,
        scratch_types=[
            pltpu.VMEM((epw,), jnp.int32),
            pltpu.VMEM((epw,), jnp.int32),
            pltpu.VMEM((_B,), jnp.int32),
            pltpu.VMEM((_B,), jnp.int32),
            pltpu.VMEM((_B,), jnp.int32),
            pltpu.VMEM((_B,), jnp.int32),
            pltpu.VMEM((_B, 2 * F), jnp.float32),
            pltpu.VMEM((_B, 2 * F), jnp.float32),
            pltpu.VMEM((_B, 2 * F), jnp.float32),
            pltpu.VMEM((_B, F), jnp.float32),
            pltpu.VMEM((_B, 2 * F), jnp.float32),
            pltpu.VMEM((_B, 2 * F), jnp.float32),
            pltpu.VMEM((_B, 2 * F), jnp.float32),
            pltpu.VMEM((_B, F), jnp.float32),
            pltpu.VMEM_SHARED((n, F), jnp.float32),
            pltpu.SemaphoreType.DMA,
            pltpu.SemaphoreType.DMA,
            pltpu.SemaphoreType.DMA,
            pltpu.SemaphoreType.DMA,
            pltpu.SemaphoreType.DMA,
            pltpu.SemaphoreType.DMA,
        ],
    )
    return kern(s, d, ea, src, dst, zeros)


def kernel(x, edge_index, edge_attr, wf1, bf1, ws1, bs1, wf2, bf2, ws2, bs2,
           wf3, bf3, ws3, bs3, w21, b21, w22, b22, w23, b23):
    n = x.shape[0]
    src = edge_index[0]
    dst = edge_index[1]
    zeros = jnp.zeros((n, F), jnp.float32)

    def mk(wf, ws, bf, bs):
        wall = jnp.concatenate(
            [wf[:F], ws[:F], wf[F:2 * F], ws[F:2 * F]], axis=1)
        brow = jnp.concatenate([bf, bs]).reshape(1, 2 * F)
        wea = jnp.concatenate([wf[2 * F:], ws[2 * F:]], axis=1)
        return wall, brow, wea

    wall1, brow1, wea1 = mk(wf1, ws1, bf1, bs1)
    wall2, brow2, wea2 = mk(wf2, ws2, bf2, bs2)
    wall3, brow3, wea3 = mk(wf3, ws3, bf3, bs3)

    ea1, ea2, ea3 = _ea_proj(
        edge_attr, jnp.concatenate([wea1, wea2, wea3], axis=1))

    s1, d1 = _proj(x, wall1, brow1)
    p1 = _sc_edge(s1, d1, ea1, src, dst, zeros)
    env2 = _update(x, p1, s1, d1)

    s2, d2 = _proj(env2, wall2, brow2)
    p2 = _sc_edge(s2, d2, ea2, src, dst, zeros)
    env3 = _update(env2, p2, s2, d2)

    s3, d3 = _proj(env3, wall3, brow3)
    p3 = _sc_edge(s3, d3, ea3, src, dst, zeros)

    return _final(env3, p3, s3, d3,
                  w21, b21.reshape(1, 2 * F),
                  w22, b22.reshape(1, F),
                  w23.reshape(1, F), b23.reshape(1, 1))


# speedup vs baseline: 1.3509x; 1.1632x over previous
import functools

import jax
import jax.numpy as jnp
from jax import lax
from jax.experimental import pallas as pl
from jax.experimental.pallas import tpu as pltpu
from jax.experimental.pallas import tpu_sc as plsc

F = 128
NC = 2
NS = 16
NW = NC * NS


def _rows(n, pref):
    return pref if n % pref == 0 else n


def _proj_body(env_ref, w_ref, b_ref, s_ref, d_ref):
    p = jnp.dot(env_ref[...], w_ref[...], preferred_element_type=jnp.float32)
    s_ref[...] = p[:, : 2 * F]
    d_ref[...] = p[:, 2 * F :] + b_ref[...]


def _proj(env, wall, brow):
    n = env.shape[0]
    r = _rows(n, 1000)
    return pl.pallas_call(
        _proj_body,
        grid=(n // r,),
        in_specs=[
            pl.BlockSpec((r, F), lambda i: (i, 0)),
            pl.BlockSpec((F, 4 * F), lambda i: (0, 0)),
            pl.BlockSpec((1, 2 * F), lambda i: (0, 0)),
        ],
        out_specs=[
            pl.BlockSpec((r, 2 * F), lambda i: (i, 0)),
            pl.BlockSpec((r, 2 * F), lambda i: (i, 0)),
        ],
        out_shape=[jax.ShapeDtypeStruct((n, 2 * F), jnp.float32)] * 2,
    )(env, wall, brow)


def _ea_body(ea_ref, w_ref, o1, o2, o3):
    p = jnp.dot(ea_ref[...], w_ref[...], preferred_element_type=jnp.float32)
    o1[...] = p[:, : 2 * F]
    o2[...] = p[:, 2 * F : 4 * F]
    o3[...] = p[:, 4 * F :]


def _ea_proj(ea, w3):
    e, k = ea.shape
    r = _rows(e, 2000)
    return pl.pallas_call(
        _ea_body,
        grid=(e // r,),
        in_specs=[
            pl.BlockSpec((r, k), lambda i: (i, 0)),
            pl.BlockSpec((k, 6 * F), lambda i: (0, 0)),
        ],
        out_specs=[pl.BlockSpec((r, 2 * F), lambda i: (i, 0))] * 3,
        out_shape=[jax.ShapeDtypeStruct((e, 2 * F), jnp.float32)] * 3,
    )(ea, w3)


def _self_msg(s, d):
    gf = s[:, :F] + d[:, :F]
    gs = s[:, F:] + d[:, F:]
    sig = 1.0 / (1.0 + jnp.exp(-gf))
    sp = jnp.maximum(gs, 0.0) + jnp.log(1.0 + jnp.exp(-jnp.abs(gs)))
    return sig * sp


def _update_body(e_ref, p_ref, s_ref, d_ref, o_ref):
    o_ref[...] = (
        e_ref[...]
        + p_ref[0]
        + p_ref[1]
        + _self_msg(s_ref[...], d_ref[...])
    )


def _update(env, part, s, d):
    n = env.shape[0]
    r = _rows(n, 1000)
    return pl.pallas_call(
        _update_body,
        grid=(n // r,),
        in_specs=[
            pl.BlockSpec((r, F), lambda i: (i, 0)),
            pl.BlockSpec((NC, r, F), lambda i: (0, i, 0)),
            pl.BlockSpec((r, 2 * F), lambda i: (i, 0)),
            pl.BlockSpec((r, 2 * F), lambda i: (i, 0)),
        ],
        out_specs=pl.BlockSpec((r, F), lambda i: (i, 0)),
        out_shape=jax.ShapeDtypeStruct((n, F), jnp.float32),
    )(env, part, s, d)


def _final_body(e_ref, p_ref, s_ref, d_ref, w21_ref, b21_ref, w22_ref,
                b22_ref, w23_ref, b23_ref, o_ref, acc_ref, *, n):
    i = pl.program_id(0)

    @pl.when(i == 0)
    def _():
        acc_ref[...] = jnp.zeros_like(acc_ref)

    env4 = (
        e_ref[...]
        + p_ref[0]
        + p_ref[1]
        + _self_msg(s_ref[...], d_ref[...])
    )
    acc_ref[...] += jnp.sum(env4, axis=0, keepdims=True)

    @pl.when(i == pl.num_programs(0) - 1)
    def _():
        pooled = acc_ref[...] * (1.0 / n)
        h1 = jnp.maximum(
            jnp.dot(pooled, w21_ref[...], preferred_element_type=jnp.float32)
            + b21_ref[...], 0.0)
        h2 = jnp.maximum(
            jnp.dot(h1, w22_ref[...], preferred_element_type=jnp.float32)
            + b22_ref[...], 0.0)
        o_ref[...] = (
            jnp.sum(h2 * w23_ref[...], axis=1, keepdims=True) + b23_ref[...]
        )


def _final(env, part, s, d, w21, b21, w22, b22, w23r, b23r):
    n = env.shape[0]
    r = _rows(n, 1000)
    return pl.pallas_call(
        functools.partial(_final_body, n=n),
        grid=(n // r,),
        in_specs=[
            pl.BlockSpec((r, F), lambda i: (i, 0)),
            pl.BlockSpec((NC, r, F), lambda i: (0, i, 0)),
            pl.BlockSpec((r, 2 * F), lambda i: (i, 0)),
            pl.BlockSpec((r, 2 * F), lambda i: (i, 0)),
            pl.BlockSpec((F, 2 * F), lambda i: (0, 0)),
            pl.BlockSpec((1, 2 * F), lambda i: (0, 0)),
            pl.BlockSpec((2 * F, F), lambda i: (0, 0)),
            pl.BlockSpec((1, F), lambda i: (0, 0)),
            pl.BlockSpec((1, F), lambda i: (0, 0)),
            pl.BlockSpec((1, 1), lambda i: (0, 0)),
        ],
        out_specs=pl.BlockSpec((1, 1), lambda i: (0, 0)),
        out_shape=jax.ShapeDtypeStruct((1, 1), jnp.float32),
        scratch_shapes=[pltpu.VMEM((1, F), jnp.float32)],
        compiler_params=pltpu.CompilerParams(
            dimension_semantics=("arbitrary",)),
    )(env, part, s, d, w21, b21, w22, b22, w23r, b23r)


_B = 16


def _gate16(bs, bd, be, bm, rr):
    c8, c7, c6, c5, c4 = (0.0051261021414032125, -0.02907406467853027,
                          0.07751608674076167, -0.13602247622393474,
                          0.19076880735651539)
    c3, c2, c1, c0 = (-0.24835398988480129, 0.3331812170752912,
                      -0.49999444976340335, 0.9999999659255092)
    for cc in range(F // 16):
        o = cc * 16
        gf = (bs[rr, pl.ds(o, 16)] + bd[rr, pl.ds(o, 16)]
              + be[rr, pl.ds(o, 16)])
        gs = (bs[rr, pl.ds(F + o, 16)] + bd[rr, pl.ds(F + o, 16)]
              + be[rr, pl.ds(F + o, 16)])
        ef = jnp.exp(-gf)
        t = jnp.exp(jnp.minimum(gs, -gs))
        p = c8
        for cx in (c7, c6, c5, c4, c3, c2, c1, c0):
            p = p * t + cx
        sp = jnp.maximum(gs, 0.0) + t * p
        bm[rr, pl.ds(o, 16)] = sp / (1.0 + ef)


def _sc_body(s_hbm, d_hbm, ea_hbm, src_hbm, dst_hbm, z_hbm, out_hbm,
             src_all, dst_all,
             bs0, bd0, be0, bm0, bs1, bd1, be1, bm1,
             acc, ss0, sd0, se0, sm0, ss1, sd1, se1, sm1, *, n, e):
    c = lax.axis_index("c")
    s = lax.axis_index("s")
    wid = s * NC + c
    epw = e // NW
    chunks = epw // _B
    rpt = (n // NS) // 8 * 8
    rem = n - NS * rpt
    r0 = pl.multiple_of(s * rpt, 8)
    pltpu.sync_copy(z_hbm.at[pl.ds(r0, rpt)], acc.at[pl.ds(r0, rpt)])
    if rem:
        @pl.when(s == 0)
        def _():
            pltpu.sync_copy(z_hbm.at[pl.ds(NS * rpt, rem)],
                            acc.at[pl.ds(NS * rpt, rem)])
    plsc.subcore_barrier()

    base = pl.multiple_of(wid * epw, 8)
    pltpu.sync_copy(src_hbm.at[pl.ds(base, epw)], src_all)
    pltpu.sync_copy(dst_hbm.at[pl.ds(base, epw)], dst_all)

    parity = ((bs0, bd0, be0, bm0, ss0, sd0, se0, sm0),
              (bs1, bd1, be1, bm1, ss1, sd1, se1, sm1))

    def issue(g, p):
        bs, bd, be, _, s_s, s_d, s_e, _ = parity[p]
        io = pl.multiple_of(g * _B, 8)
        pltpu.async_copy(s_hbm.at[src_all[pl.ds(io, _B)]], bs, s_s)
        pltpu.async_copy(d_hbm.at[dst_all[pl.ds(io, _B)]], bd, s_d)
        off = pl.multiple_of(base + g * _B, 8)
        pltpu.async_copy(ea_hbm.at[pl.ds(off, _B)], be, s_e)

    def consume(g, p):
        bs, bd, be, bm, s_s, s_d, s_e, s_m = parity[p]
        io = pl.multiple_of(g * _B, 8)
        pltpu.make_async_copy(s_hbm.at[src_all[pl.ds(io, _B)]], bs, s_s).wait()
        pltpu.make_async_copy(d_hbm.at[dst_all[pl.ds(io, _B)]], bd, s_d).wait()
        pltpu.make_async_copy(ea_hbm.at[pl.ds(0, _B)], be, s_e).wait()

        @pl.when(g >= 2)
        def _():
            pltpu.make_async_copy(
                bm, acc.at[dst_all[pl.ds(0, _B)]], s_m).wait()

        def _row(rr, carry2):
            _gate16(bs, bd, be, bm, rr)
            return carry2

        lax.fori_loop(0, _B, _row, 0)

        pltpu.async_copy(bm, acc.at[dst_all[pl.ds(io, _B)]], s_m, add=True)

    issue(0, 0)

    def pair(k, carry):
        issue(2 * k + 1, 1)
        consume(2 * k, 0)
        issue(2 * k + 2, 0)
        consume(2 * k + 1, 1)
        return carry

    lax.fori_loop(0, (chunks - 1) // 2, pair, 0)
    consume(chunks - 1, 0)
    pltpu.make_async_copy(bm1, acc.at[dst_all[pl.ds(0, _B)]], sm1).wait()
    pltpu.make_async_copy(bm0, acc.at[dst_all[pl.ds(0, _B)]], sm0).wait()

    plsc.subcore_barrier()
    pltpu.sync_copy(acc.at[pl.ds(r0, rpt)], out_hbm.at[c, pl.ds(r0, rpt)])
    if rem:
        @pl.when(s == 0)
        def _():
            pltpu.sync_copy(acc.at[pl.ds(NS * rpt, rem)],
                            out_hbm.at[c, pl.ds(NS * rpt, rem)])


def _sc_edge(s, d, ea, src, dst, zeros):
    n = s.shape[0]
    e = src.shape[0]
    epw = e // NW
    mesh = plsc.VectorSubcoreMesh(
        core_axis_name="c", subcore_axis_name="s",
        num_cores=NC, num_subcores=NS)
    kern = pl.kernel(
        functools.partial(_sc_body, n=n, e=e),
        out_type=jax.ShapeDtypeStruct((NC, n, F), jnp.float32),
        mesh=mesh,
        scratch_types=[
            pltpu.VMEM((epw,), jnp.int32),
            pltpu.VMEM((epw,), jnp.int32),
            pltpu.VMEM((_B, 2 * F), jnp.float32),
            pltpu.VMEM((_B, 2 * F), jnp.float32),
            pltpu.VMEM((_B, 2 * F), jnp.float32),
            pltpu.VMEM((_B, F), jnp.float32),
            pltpu.VMEM((_B, 2 * F), jnp.float32),
            pltpu.VMEM((_B, 2 * F), jnp.float32),
            pltpu.VMEM((_B, 2 * F), jnp.float32),
            pltpu.VMEM((_B, F), jnp.float32),
            pltpu.VMEM_SHARED((n, F), jnp.float32),
            pltpu.SemaphoreType.DMA,
            pltpu.SemaphoreType.DMA,
            pltpu.SemaphoreType.DMA,
            pltpu.SemaphoreType.DMA,
            pltpu.SemaphoreType.DMA,
            pltpu.SemaphoreType.DMA,
            pltpu.SemaphoreType.DMA,
            pltpu.SemaphoreType.DMA,
        ],
    )
    return kern(s, d, ea, src, dst, zeros)


def kernel(x, edge_index, edge_attr, wf1, bf1, ws1, bs1, wf2, bf2, ws2, bs2,
           wf3, bf3, ws3, bs3, w21, b21, w22, b22, w23, b23):
    n = x.shape[0]
    src = edge_index[0]
    dst = edge_index[1]
    zeros = jnp.zeros((n, F), jnp.float32)

    def mk(wf, ws, bf, bs):
        wall = jnp.concatenate(
            [wf[:F], ws[:F], wf[F:2 * F], ws[F:2 * F]], axis=1)
        brow = jnp.concatenate([bf, bs]).reshape(1, 2 * F)
        wea = jnp.concatenate([wf[2 * F:], ws[2 * F:]], axis=1)
        return wall, brow, wea

    wall1, brow1, wea1 = mk(wf1, ws1, bf1, bs1)
    wall2, brow2, wea2 = mk(wf2, ws2, bf2, bs2)
    wall3, brow3, wea3 = mk(wf3, ws3, bf3, bs3)

    ea1, ea2, ea3 = _ea_proj(
        edge_attr, jnp.concatenate([wea1, wea2, wea3], axis=1))

    s1, d1 = _proj(x, wall1, brow1)
    p1 = _sc_edge(s1, d1, ea1, src, dst, zeros)
    env2 = _update(x, p1, s1, d1)

    s2, d2 = _proj(env2, wall2, brow2)
    p2 = _sc_edge(s2, d2, ea2, src, dst, zeros)
    env3 = _update(env2, p2, s2, d2)

    s3, d3 = _proj(env3, wall3, brow3)
    p3 = _sc_edge(s3, d3, ea3, src, dst, zeros)

    return _final(env3, p3, s3, d3,
                  w21, b21.reshape(1, 2 * F),
                  w22, b22.reshape(1, F),
                  w23.reshape(1, F), b23.reshape(1, 1))
